# baseline (speedup 1.0000x reference)
import functools

import jax
import jax.numpy as jnp
from jax import lax
from jax.experimental import pallas as pl
from jax.experimental.pallas import tpu as pltpu
from jax.experimental.pallas import tpu_sc as plsc

VOCAB = 1000000
EMB = 64
OUT = 128
B = 16384
L = 200
LH = L // 2
NB = 64
LANES = 16

_info = plsc.get_sparse_core_info()
NC, NS = _info.num_cores, _info.num_subcores
NW = NC * NS
BAGS_PER_W = B // NW
CHUNKS = BAGS_PER_W // NB


def _sc_body(ids_hbm, table_hbm, sums_hbm, idx_v,
             rows_a0, rows_a1, rows_b0, rows_b1, out_v,
             sem_a0, sem_a1, sem_b0, sem_b1):
    wid = lax.axis_index("s") * NC + lax.axis_index("c")

    def issue(i, r0, r1, s0, s1):
        pltpu.async_copy(table_hbm.at[idx_v.at[i, 0]], r0, s0)
        pltpu.async_copy(table_hbm.at[idx_v.at[i, 1]], r1, s1)

    def wait(r0, r1, s0, s1):
        pltpu.make_async_copy(table_hbm.at[idx_v.at[0, 0]], r0, s0).wait()
        pltpu.make_async_copy(table_hbm.at[idx_v.at[0, 1]], r1, s1).wait()

    def reduce_into(i, r0, r1):
        msk = jnp.full((LANES,), -65536, jnp.int32)

        def red_body(r, acc):
            a0, a1, a2, a3 = acc
            v0 = plsc.bitcast(r0[r, pl.ds(0, LANES)], jnp.int32)
            v1 = plsc.bitcast(r0[r, pl.ds(LANES, LANES)], jnp.int32)
            v2 = plsc.bitcast(r1[r, pl.ds(0, LANES)], jnp.int32)
            v3 = plsc.bitcast(r1[r, pl.ds(LANES, LANES)], jnp.int32)
            a0 = (a0 + plsc.bitcast(lax.shift_left(v0, 16), jnp.float32)
                  + plsc.bitcast(lax.shift_left(v2, 16), jnp.float32))
            a1 = (a1 + plsc.bitcast(lax.bitwise_and(v0, msk), jnp.float32)
                  + plsc.bitcast(lax.bitwise_and(v2, msk), jnp.float32))
            a2 = (a2 + plsc.bitcast(lax.shift_left(v1, 16), jnp.float32)
                  + plsc.bitcast(lax.shift_left(v3, 16), jnp.float32))
            a3 = (a3 + plsc.bitcast(lax.bitwise_and(v1, msk), jnp.float32)
                  + plsc.bitcast(lax.bitwise_and(v3, msk), jnp.float32))
            return (a0, a1, a2, a3)

        acc = lax.fori_loop(
            0, LH, red_body,
            tuple(jnp.zeros((LANES,), jnp.float32)
                  for _ in range(EMB // LANES)),
            unroll=2)
        for j in range(EMB // LANES):
            out_v[i, pl.ds(j * LANES, LANES)] = acc[j]

    def chunk_body(ci, _):
        base = wid * BAGS_PER_W + ci * NB
        pltpu.sync_copy(ids_hbm.at[pl.ds(base, NB)], idx_v)
        issue(0, rows_a0, rows_a1, sem_a0, sem_a1)

        def pair_body(k, _):
            i0 = 2 * k
            issue(i0 + 1, rows_b0, rows_b1, sem_b0, sem_b1)
            wait(rows_a0, rows_a1, sem_a0, sem_a1)
            reduce_into(i0, rows_a0, rows_a1)

            @pl.when(k < NB // 2 - 1)
            def _():
                issue(i0 + 2, rows_a0, rows_a1, sem_a0, sem_a1)

            wait(rows_b0, rows_b1, sem_b0, sem_b1)
            reduce_into(i0 + 1, rows_b0, rows_b1)
            return ()

        lax.fori_loop(0, NB // 2, pair_body, ())
        pltpu.sync_copy(out_v, sums_hbm.at[pl.ds(base, NB)])
        return ()

    lax.fori_loop(0, CHUNKS, chunk_body, ())


_sc_pool = functools.partial(
    pl.kernel,
    out_type=jax.ShapeDtypeStruct((B, EMB), jnp.float32),
    mesh=plsc.VectorSubcoreMesh(core_axis_name="c", subcore_axis_name="s"),
    scratch_types=[
        pltpu.VMEM((NB, 2, LH), jnp.int32),
        pltpu.VMEM((LH, EMB // 2), jnp.float32),
        pltpu.VMEM((LH, EMB // 2), jnp.float32),
        pltpu.VMEM((LH, EMB // 2), jnp.float32),
        pltpu.VMEM((LH, EMB // 2), jnp.float32),
        pltpu.VMEM((NB, EMB), jnp.float32),
        pltpu.SemaphoreType.DMA,
        pltpu.SemaphoreType.DMA,
        pltpu.SemaphoreType.DMA,
        pltpu.SemaphoreType.DMA,
    ],
    compiler_params=pltpu.CompilerParams(use_tc_tiling_on_sc=False,
                                         needs_layout_passes=False),
)(_sc_body)


def _tc_matmul_body(sums_ref, w_ref, b_ref, out_ref):
    out_ref[...] = (
        lax.dot_general(sums_ref[...], w_ref[...], (((1,), (1,)), ((), ())),
                        preferred_element_type=jnp.float32)
        * (1.0 / L)
        + b_ref[...]
    )


def _tc_matmul(sums, w, b):
    blk = 2048
    return pl.pallas_call(
        _tc_matmul_body,
        grid=(B // blk,),
        in_specs=[
            pl.BlockSpec((blk, EMB), lambda i: (i, 0)),
            pl.BlockSpec((OUT, EMB), lambda i: (0, 0)),
            pl.BlockSpec((1, OUT), lambda i: (0, 0)),
        ],
        out_specs=pl.BlockSpec((blk, OUT), lambda i: (i, 0)),
        out_shape=jax.ShapeDtypeStruct((B, OUT), jnp.float32),
    )(sums, w, b)


_PERM = (list(range(0, 16)) + list(range(32, 48))
         + list(range(16, 32)) + list(range(48, 64)))


def kernel(ids, length, emb_table, W, b):
    del length
    ids32 = ids.astype(jnp.int32).reshape(B, 2, LH)
    ti = jax.lax.bitcast_convert_type(emb_table, jnp.int32)
    rnd = ti + jnp.int32(0x7FFF) + jnp.bitwise_and(
        jax.lax.shift_right_logical(ti, 16), jnp.int32(1))
    hi16 = jax.lax.shift_right_logical(rnd, 16)
    packed_i = jnp.bitwise_or(hi16[:, : EMB // 2],
                              jax.lax.shift_left(hi16[:, EMB // 2:], 16))
    packed = jax.lax.bitcast_convert_type(packed_i, jnp.float32)
    sums = _sc_pool(ids32, packed)
    return _tc_matmul(sums, W[:, jnp.array(_PERM)], b.reshape(1, OUT))


# speedup vs baseline: 1.7034x; 1.7034x over previous
import functools

import jax
import jax.numpy as jnp
from jax import lax
from jax.experimental import pallas as pl
from jax.experimental.pallas import tpu as pltpu
from jax.experimental.pallas import tpu_sc as plsc

VOCAB = 1000000
EMB = 64
OUT = 128
B = 16384
L = 200
LH = L // 2
NB = 64
LANES = 16

_info = plsc.get_sparse_core_info()
NC, NS = _info.num_cores, _info.num_subcores
NW = NC * NS
BAGS_PER_W = B // NW
CHUNKS = BAGS_PER_W // NB


def _sc_body(ids_hbm, table_hbm, sums_hbm, idx_v,
             rows_a0, rows_a1, rows_b0, rows_b1, out_v,
             sem_a0, sem_a1, sem_b0, sem_b1):
    wid = lax.axis_index("s") * NC + lax.axis_index("c")

    def issue(i, r0, r1, s0, s1):
        pltpu.async_copy(table_hbm.at[idx_v.at[i, 0]], r0, s0)
        pltpu.async_copy(table_hbm.at[idx_v.at[i, 1]], r1, s1)

    def wait(r0, r1, s0, s1):
        pltpu.make_async_copy(table_hbm.at[idx_v.at[0, 0]], r0, s0).wait()
        pltpu.make_async_copy(table_hbm.at[idx_v.at[0, 1]], r1, s1).wait()

    def reduce_into(i, r0, r1):
        def red_body(r, acc):
            return tuple(
                acc[j]
                + r0[r, pl.ds(j * LANES, LANES)]
                + r1[r, pl.ds(j * LANES, LANES)]
                for j in range(EMB // LANES)
            )

        acc = lax.fori_loop(
            0, LH, red_body,
            tuple(jnp.zeros((LANES,), jnp.float32)
                  for _ in range(EMB // LANES)),
            unroll=2)
        for j in range(EMB // LANES):
            out_v[i, pl.ds(j * LANES, LANES)] = acc[j]

    def chunk_body(ci, _):
        base = wid * BAGS_PER_W + ci * NB
        pltpu.sync_copy(ids_hbm.at[pl.ds(base, NB)], idx_v)
        issue(0, rows_a0, rows_a1, sem_a0, sem_a1)

        def pair_body(k, _):
            i0 = 2 * k
            issue(i0 + 1, rows_b0, rows_b1, sem_b0, sem_b1)
            wait(rows_a0, rows_a1, sem_a0, sem_a1)
            reduce_into(i0, rows_a0, rows_a1)

            @pl.when(k < NB // 2 - 1)
            def _():
                issue(i0 + 2, rows_a0, rows_a1, sem_a0, sem_a1)

            wait(rows_b0, rows_b1, sem_b0, sem_b1)
            reduce_into(i0 + 1, rows_b0, rows_b1)
            return ()

        lax.fori_loop(0, NB // 2, pair_body, ())
        pltpu.sync_copy(out_v, sums_hbm.at[pl.ds(base, NB)])
        return ()

    lax.fori_loop(0, CHUNKS, chunk_body, ())


_sc_pool = functools.partial(
    pl.kernel,
    out_type=jax.ShapeDtypeStruct((B, EMB), jnp.float32),
    mesh=plsc.VectorSubcoreMesh(core_axis_name="c", subcore_axis_name="s"),
    scratch_types=[
        pltpu.VMEM((NB, 2, LH), jnp.int32),
        pltpu.VMEM((LH, EMB), jnp.float32),
        pltpu.VMEM((LH, EMB), jnp.float32),
        pltpu.VMEM((LH, EMB), jnp.float32),
        pltpu.VMEM((LH, EMB), jnp.float32),
        pltpu.VMEM((NB, EMB), jnp.float32),
        pltpu.SemaphoreType.DMA,
        pltpu.SemaphoreType.DMA,
        pltpu.SemaphoreType.DMA,
        pltpu.SemaphoreType.DMA,
    ],
    compiler_params=pltpu.CompilerParams(use_tc_tiling_on_sc=False,
                                         needs_layout_passes=False),
)(_sc_body)


def _tc_matmul_body(sums_ref, w_ref, b_ref, out_ref):
    out_ref[...] = (
        lax.dot_general(sums_ref[...], w_ref[...], (((1,), (1,)), ((), ())),
                        preferred_element_type=jnp.float32)
        * (1.0 / L)
        + b_ref[...]
    )


def _tc_matmul(sums, w, b):
    blk = 2048
    return pl.pallas_call(
        _tc_matmul_body,
        grid=(B // blk,),
        in_specs=[
            pl.BlockSpec((blk, EMB), lambda i: (i, 0)),
            pl.BlockSpec((OUT, EMB), lambda i: (0, 0)),
            pl.BlockSpec((1, OUT), lambda i: (0, 0)),
        ],
        out_specs=pl.BlockSpec((blk, OUT), lambda i: (i, 0)),
        out_shape=jax.ShapeDtypeStruct((B, OUT), jnp.float32),
    )(sums, w, b)


def kernel(ids, length, emb_table, W, b):
    del length
    ids32 = ids.astype(jnp.int32).reshape(B, 2, LH)
    sums = _sc_pool(ids32, emb_table)
    return _tc_matmul(sums, W, b.reshape(1, OUT))
